# baseline (speedup 1.0000x reference)
import functools

import jax
import jax.numpy as jnp
from jax import lax
from jax.experimental import pallas as pl
from jax.experimental.pallas import tpu as pltpu
from jax.experimental.pallas import tpu_sc as plsc

_NUM_CORES = 2
_NUM_SUBCORES = 16
_NUM_WORKERS = _NUM_CORES * _NUM_SUBCORES
_CHUNK = 32
_NBUF = 2


@functools.partial(jax.jit, static_argnums=(2,))
def _sc_gather(ids, table, n_per_w):
    d = table.shape[1]
    s = ids.shape[1]
    n = ids.shape[0] * s
    n_chunks = n_per_w // _CHUNK
    w_per_row = s // n_per_w
    mesh = plsc.VectorSubcoreMesh(core_axis_name="c", subcore_axis_name="s")

    @functools.partial(
        pl.kernel,
        mesh=mesh,
        out_type=jax.ShapeDtypeStruct((n, d), table.dtype),
        scratch_types=[
            pltpu.VMEM((n_per_w,), jnp.int32),
            pltpu.SemaphoreType.DMA,
        ],
    )
    def k(ids_hbm, table_hbm, out_hbm, idx_s, sem):
        wid = lax.axis_index("s") * _NUM_CORES + lax.axis_index("c")
        base = wid * n_per_w
        out_w = out_hbm.at[pl.ds(base, n_per_w)]
        pltpu.sync_copy(
            ids_hbm.at[wid // w_per_row, pl.ds((wid % w_per_row) * n_per_w, n_per_w)],
            idx_s,
        )
        n_groups = n_per_w // 16

        @pl.loop(0, n_groups)
        def _(g):
            vec = idx_s[pl.ds(g * 16, 16)]
            for j in range(16):
                pltpu.async_copy(
                    table_hbm.at[pl.ds(vec[j], 1)],
                    out_w.at[pl.ds(g * 16 + j, 1)],
                    sem,
                )

            @pl.when(g >= 1)
            def _():
                for j in range(16):
                    pltpu.make_async_copy(
                        table_hbm.at[pl.ds(0, 1)], out_w.at[pl.ds(0, 1)], sem
                    ).wait()

        for j in range(16):
            pltpu.make_async_copy(
                table_hbm.at[pl.ds(0, 1)], out_w.at[pl.ds(0, 1)], sem
            ).wait()

    return k(ids, table)


def kernel(input_ids, embed_table):
    b, s = input_ids.shape
    n_per_w = (b * s) // _NUM_WORKERS
    out = _sc_gather(input_ids.astype(jnp.int32), embed_table, n_per_w)
    return out.reshape(b, s, embed_table.shape[1])


# speedup vs baseline: 35.2547x; 35.2547x over previous
import functools

import jax
import jax.numpy as jnp
from jax import lax
from jax.experimental import pallas as pl
from jax.experimental.pallas import tpu as pltpu
from jax.experimental.pallas import tpu_sc as plsc

_IDXBLK = 128
_WINDOW = 32
_SUBS = _IDXBLK // _WINDOW


@jax.jit
def _sc_gather(ids, table):
    b, s = ids.shape
    n = b * s
    d = table.shape[1]
    blocks_per_row = s // _IDXBLK
    mesh = plsc.VectorSubcoreMesh(core_axis_name="c", subcore_axis_name="s")

    @functools.partial(
        pl.kernel,
        mesh=mesh,
        out_type=jax.ShapeDtypeStruct((n, d), table.dtype),
    )
    def k(ids_hbm, table_hbm, out_hbm):
        def body(i_vmem, o_vmem):
            sub = lax.rem(pl.program_id(0), _SUBS)
            pltpu.sync_copy(
                table_hbm.at[i_vmem.at[0, pl.ds(sub * _WINDOW, _WINDOW)]],
                o_vmem,
            )

        pltpu.emit_pipeline(
            body,
            grid=(n // _WINDOW,),
            in_specs=[
                pl.BlockSpec(
                    (1, _IDXBLK),
                    lambda i: (
                        (i // _SUBS) // blocks_per_row,
                        (i // _SUBS) % blocks_per_row,
                    ),
                )
            ],
            out_specs=[pl.BlockSpec((_WINDOW, d), lambda i: (i, 0))],
            core_axis_name=("c", "s"),
            dimension_semantics=(pltpu.PARALLEL,),
        )(ids_hbm, out_hbm)

    return k(ids, table)


def kernel(input_ids, embed_table):
    b, s = input_ids.shape
    out = _sc_gather(input_ids.astype(jnp.int32), embed_table)
    return out.reshape(b, s, embed_table.shape[1])


# speedup vs baseline: 36.3793x; 1.0319x over previous
import functools

import jax
import jax.numpy as jnp
from jax import lax
from jax.experimental import pallas as pl
from jax.experimental.pallas import tpu as pltpu
from jax.experimental.pallas import tpu_sc as plsc

_NUM_CORES = 2
_NUM_SUBCORES = 16
_NUM_WORKERS = _NUM_CORES * _NUM_SUBCORES
_CHUNK = 32
_NBUF = 2


@functools.partial(jax.jit, static_argnums=(2,))
def _sc_gather(ids, table, n_per_w):
    d = table.shape[1]
    s = ids.shape[1]
    n = ids.shape[0] * s
    n_chunks = n_per_w // _CHUNK
    w_per_row = s // n_per_w
    mesh = plsc.VectorSubcoreMesh(core_axis_name="c", subcore_axis_name="s")

    @functools.partial(
        pl.kernel,
        mesh=mesh,
        out_type=jax.ShapeDtypeStruct((n, d), table.dtype),
        scratch_types=[
            pltpu.VMEM((n_per_w,), jnp.int32),
            pltpu.VMEM((_NBUF, _CHUNK, d), table.dtype),
            pltpu.SemaphoreType.DMA,
            pltpu.SemaphoreType.DMA,
            pltpu.SemaphoreType.DMA,
            pltpu.SemaphoreType.DMA,
        ],
    )
    def k(ids_hbm, table_hbm, out_hbm, idx_v, rows_v, g0, g1, o0, o1):
        gsems = (g0, g1)
        osems = (o0, o1)
        wid = lax.axis_index("s") * _NUM_CORES + lax.axis_index("c")
        base = wid * n_per_w
        out_w = out_hbm.at[pl.ds(base, n_per_w)]
        pltpu.sync_copy(
            ids_hbm.at[wid // w_per_row, pl.ds((wid % w_per_row) * n_per_w, n_per_w)],
            idx_v,
        )

        for b in range(_NBUF):
            pltpu.async_copy(
                table_hbm.at[idx_v.at[pl.ds(b * _CHUNK, _CHUNK)]],
                rows_v.at[b],
                gsems[b],
            )

        @pl.loop(0, n_chunks, step=_NBUF)
        def _(c0):
            for b in range(_NBUF):
                c = c0 + b
                pltpu.make_async_copy(
                    table_hbm.at[pl.ds(0, _CHUNK)], rows_v.at[b], gsems[b]
                ).wait()
                pltpu.async_copy(
                    rows_v.at[b], out_w.at[pl.ds(c * _CHUNK, _CHUNK)], osems[b]
                ).wait()
                nxt = c + _NBUF

                @pl.when(nxt < n_chunks)
                def _():
                    pltpu.async_copy(
                        table_hbm.at[idx_v.at[pl.ds(nxt * _CHUNK, _CHUNK)]],
                        rows_v.at[b],
                        gsems[b],
                    )

    return k(ids, table)


def kernel(input_ids, embed_table):
    b, s = input_ids.shape
    n_per_w = (b * s) // _NUM_WORKERS
    out = _sc_gather(input_ids.astype(jnp.int32), embed_table, n_per_w)
    return out.reshape(b, s, embed_table.shape[1])
